# baseline (speedup 1.0000x reference)
import functools
import jax
import jax.numpy as jnp
from jax import lax
from jax.experimental import pallas as pl
from jax.experimental.pallas import tpu as pltpu
from jax.experimental.pallas import tpu_sc as plsc

N = 10000
P = 10240
E = 320000
D = 128
EPS = 1e-5

NC, NS, L = 2, 16, 16
NW = NC * NS
EW = E // NW
CH = 50
NCH = EW // CH
IBLK = 25
NBLK = NCH // IBLK
RPT = P // NS

@functools.cache
def _mesh():
    return plsc.VectorSubcoreMesh(
        core_axis_name="c", subcore_axis_name="s", num_cores=NC, num_subcores=NS
    )


def _deg_kernel_body(dst_hbm, out_hbm, dst_v, hist_v, sem):
    cid = lax.axis_index("c")
    sid = lax.axis_index("s")
    wid = sid * NC + cid
    pltpu.async_copy(dst_hbm.at[pl.ds(wid * EW, EW)], dst_v, sem)

    def zero_body(i, _):
        hist_v[pl.ds(i * L, L)] = jnp.zeros((L,), jnp.float32)
        return 0

    lax.fori_loop(0, P // L, zero_body, 0)
    pltpu.make_async_copy(dst_hbm.at[pl.ds(wid * EW, EW)], dst_v, sem).wait()

    ones = jnp.ones((L,), jnp.float32)

    def body(i, _):
        idx = dst_v[pl.ds(i * L, L)]
        plsc.addupdate_scatter(hist_v, [idx], ones)
        return 0

    lax.fori_loop(0, EW // L, body, 0)
    pltpu.sync_copy(hist_v, out_hbm.at[wid])


@functools.cache
def _deg_kernel():
    return pl.kernel(
        _deg_kernel_body,
        out_type=jax.ShapeDtypeStruct((NW, P), jnp.float32),
        mesh=_mesh(),
        scratch_types=[
            pltpu.VMEM((EW,), jnp.int32),
            pltpu.VMEM((P,), jnp.float32),
            pltpu.SemaphoreType.DMA,
        ],
        compiler_params=pltpu.CompilerParams(needs_layout_passes=False),
    )


def _prescale_body(x_ref, w_ref, hist_ref, hp_ref, dinv_ref):
    deg = jnp.sum(hist_ref[...], axis=0) + 1.0
    dinv = lax.rsqrt(deg)
    xp = jnp.concatenate(
        [x_ref[...], jnp.zeros((P - N, D), jnp.float32)], axis=0)
    h = jnp.dot(xp, w_ref[...], preferred_element_type=jnp.float32)
    h3 = h.reshape(P // D, D, D) * dinv[:, :, None]
    hp_ref[...] = h3.reshape(P, D)
    dinv_ref[...] = dinv


def _prescale(x, w, hist):
    return pl.pallas_call(
        _prescale_body,
        out_shape=(
            jax.ShapeDtypeStruct((P, D), jnp.float32),
            jax.ShapeDtypeStruct((P // D, D), jnp.float32),
        ),
    )(x, w, hist)


NB = 5


def _scatter_kernel_body(hp_hbm, src_hbm, dst_hbm, zeros_hbm, out_hbm,
                         idx_src, idx_dst, *rest):
    rows = rest[:NB]
    gsems = rest[NB:2 * NB]
    acc_sh = rest[2 * NB]
    cid = lax.axis_index("c")
    sid = lax.axis_index("s")
    wid = sid * NC + cid

    row0 = sid * RPT

    @pl.when(cid == 0)
    def _():
        pltpu.sync_copy(hp_hbm.at[pl.ds(row0, RPT)],
                        acc_sh.at[pl.ds(row0, RPT)])

    @pl.when(cid != 0)
    def _():
        pltpu.sync_copy(zeros_hbm, acc_sh.at[pl.ds(row0, RPT)])

    plsc.subcore_barrier()

    def block(blk, _):
        pltpu.sync_copy(src_hbm.at[wid, blk], idx_src)
        pltpu.sync_copy(dst_hbm.at[wid, blk], idx_dst)
        for j in range(NB):
            pltpu.async_copy(hp_hbm.at[idx_src.at[j]], rows[j], gsems[j])

        def group(g, _):
            for j in range(NB):
                m = g * NB + j
                pltpu.make_async_copy(hp_hbm.at[idx_src.at[m]], rows[j],
                                      gsems[j]).wait()
                pltpu.sync_copy(rows[j], acc_sh.at[idx_dst.at[m]], add=True)
                n = m + NB

                @pl.when(n < IBLK)
                def _():
                    pltpu.async_copy(hp_hbm.at[idx_src.at[n]], rows[j],
                                     gsems[j])
            return 0

        lax.fori_loop(0, IBLK // NB, group, 0)
        return 0

    lax.fori_loop(0, NBLK, block, 0)
    plsc.subcore_barrier()

    pltpu.sync_copy(acc_sh.at[pl.ds(sid * RPT, RPT)],
                    out_hbm.at[cid, pl.ds(sid * RPT, RPT)])


@functools.cache
def _scatter_kernel():
    return pl.kernel(
        _scatter_kernel_body,
        out_type=jax.ShapeDtypeStruct((NC, P, D), jnp.float32),
        mesh=_mesh(),
        scratch_types=(
            [pltpu.VMEM((IBLK, CH), jnp.int32),
             pltpu.VMEM((IBLK, CH), jnp.int32)]
            + [pltpu.VMEM((CH, D), jnp.float32) for _ in range(NB)]
            + [pltpu.SemaphoreType.DMA for _ in range(NB)]
            + [pltpu.VMEM_SHARED((P, D), jnp.float32)]
        ),
        compiler_params=pltpu.CompilerParams(needs_layout_passes=False),
    )


def _bn_body(p_ref, dinv_ref, b_ref, gamma_ref, beta_ref, out_ref):
    s = p_ref[0] + p_ref[1]
    s3 = s.reshape(P // D, D, D) * dinv_ref[...][:, :, None]
    pre = s3.reshape(P, D) + b_ref[...]
    pre = pre[:N]
    inv_n = 1.0 / float(N)
    mean = jnp.sum(pre, axis=0, keepdims=True) * inv_n
    var = jnp.sum(pre * pre, axis=0, keepdims=True) * inv_n - mean * mean
    scale = lax.rsqrt(var + EPS) * gamma_ref[...]
    out = (pre - mean) * scale + beta_ref[...]
    out_ref[...] = jnp.maximum(out, 0.0)


def _bn(parts, dinv, b, gamma, beta):
    return pl.pallas_call(
        _bn_body,
        out_shape=jax.ShapeDtypeStruct((N, D), jnp.float32),
    )(parts, dinv, b, gamma, beta)


def kernel(x, edge_index, batch, W, b, gamma, beta):
    src = edge_index[0].astype(jnp.int32)
    dst = edge_index[1].astype(jnp.int32)

    hist = _deg_kernel()(dst).reshape(NW, P // D, D)
    hp, dinv = _prescale(x, W, hist)
    zeros_blk = jnp.zeros((RPT, D), jnp.float32)
    src4 = src.reshape(NW, NBLK, IBLK, CH)
    dst4 = dst.reshape(NW, NBLK, IBLK, CH)
    parts = _scatter_kernel()(hp, src4, dst4, zeros_blk)
    return _bn(parts, dinv, b.reshape(1, D), gamma.reshape(1, D),
               beta.reshape(1, D))


# speedup vs baseline: 1.0564x; 1.0564x over previous
import functools
import jax
import jax.numpy as jnp
from jax import lax
from jax.experimental import pallas as pl
from jax.experimental.pallas import tpu as pltpu
from jax.experimental.pallas import tpu_sc as plsc

N = 10000
P = 10240
E = 320000
D = 128
EPS = 1e-5

NC, NS, L = 2, 16, 16
NW = NC * NS
EW = E // NW
CH = 40
NCH = EW // CH
IBLK = 50
NBLK = NCH // IBLK
RPT = P // NS

@functools.cache
def _mesh():
    return plsc.VectorSubcoreMesh(
        core_axis_name="c", subcore_axis_name="s", num_cores=NC, num_subcores=NS
    )


def _deg_kernel_body(dst_hbm, out_hbm, dst_v, hist_v, sem):
    cid = lax.axis_index("c")
    sid = lax.axis_index("s")
    wid = sid * NC + cid
    pltpu.async_copy(dst_hbm.at[pl.ds(wid * EW, EW)], dst_v, sem)

    def zero_body(i, _):
        hist_v[pl.ds(i * L, L)] = jnp.zeros((L,), jnp.float32)
        return 0

    lax.fori_loop(0, P // L, zero_body, 0)
    pltpu.make_async_copy(dst_hbm.at[pl.ds(wid * EW, EW)], dst_v, sem).wait()

    ones = jnp.ones((L,), jnp.float32)

    def body(i, _):
        idx = dst_v[pl.ds(i * L, L)]
        plsc.addupdate_scatter(hist_v, [idx], ones)
        return 0

    lax.fori_loop(0, EW // L, body, 0)
    pltpu.sync_copy(hist_v, out_hbm.at[wid])


@functools.cache
def _deg_kernel():
    return pl.kernel(
        _deg_kernel_body,
        out_type=jax.ShapeDtypeStruct((NW, P), jnp.float32),
        mesh=_mesh(),
        scratch_types=[
            pltpu.VMEM((EW,), jnp.int32),
            pltpu.VMEM((P,), jnp.float32),
            pltpu.SemaphoreType.DMA,
        ],
        compiler_params=pltpu.CompilerParams(needs_layout_passes=False),
    )


def _prescale_body(x_ref, w_ref, hist_ref, hp_ref, dinv_ref):
    deg = jnp.sum(hist_ref[...], axis=0) + 1.0
    dinv = lax.rsqrt(deg)
    xp = jnp.concatenate(
        [x_ref[...], jnp.zeros((P - N, D), jnp.float32)], axis=0)
    h = jnp.dot(xp, w_ref[...], preferred_element_type=jnp.float32)
    h3 = h.reshape(P // D, D, D) * dinv[:, :, None]
    hp_ref[...] = h3.reshape(P, D)
    dinv_ref[...] = dinv


def _prescale(x, w, hist):
    return pl.pallas_call(
        _prescale_body,
        out_shape=(
            jax.ShapeDtypeStruct((P, D), jnp.float32),
            jax.ShapeDtypeStruct((P // D, D), jnp.float32),
        ),
    )(x, w, hist)


NB = 5


def _scatter_kernel_body(hp_hbm, src_hbm, dst_hbm, zeros_hbm, out_hbm,
                         idx_src, idx_dst, *rest):
    rows = rest[:NB]
    gsems = rest[NB:2 * NB]
    acc_sh = rest[2 * NB]
    cid = lax.axis_index("c")
    sid = lax.axis_index("s")
    wid = sid * NC + cid

    row0 = sid * RPT

    @pl.when(cid == 0)
    def _():
        pltpu.sync_copy(hp_hbm.at[pl.ds(row0, RPT)],
                        acc_sh.at[pl.ds(row0, RPT)])

    @pl.when(cid != 0)
    def _():
        pltpu.sync_copy(zeros_hbm, acc_sh.at[pl.ds(row0, RPT)])

    plsc.subcore_barrier()

    def block(blk, _):
        pltpu.sync_copy(src_hbm.at[wid, blk], idx_src)
        pltpu.sync_copy(dst_hbm.at[wid, blk], idx_dst)
        for j in range(NB):
            pltpu.async_copy(hp_hbm.at[idx_src.at[j]], rows[j], gsems[j])

        def group(g, _):
            for j in range(NB):
                m = g * NB + j
                pltpu.make_async_copy(hp_hbm.at[idx_src.at[m]], rows[j],
                                      gsems[j]).wait()
                pltpu.sync_copy(rows[j], acc_sh.at[idx_dst.at[m]], add=True)
                n = m + NB

                @pl.when(n < IBLK)
                def _():
                    pltpu.async_copy(hp_hbm.at[idx_src.at[n]], rows[j],
                                     gsems[j])
            return 0

        lax.fori_loop(0, IBLK // NB, group, 0)
        return 0

    lax.fori_loop(0, NBLK, block, 0)
    plsc.subcore_barrier()

    pltpu.sync_copy(acc_sh.at[pl.ds(sid * RPT, RPT)],
                    out_hbm.at[cid, pl.ds(sid * RPT, RPT)])


@functools.cache
def _scatter_kernel():
    return pl.kernel(
        _scatter_kernel_body,
        out_type=jax.ShapeDtypeStruct((NC, P, D), jnp.float32),
        mesh=_mesh(),
        scratch_types=(
            [pltpu.VMEM((IBLK, CH), jnp.int32),
             pltpu.VMEM((IBLK, CH), jnp.int32)]
            + [pltpu.VMEM((CH, D), jnp.float32) for _ in range(NB)]
            + [pltpu.SemaphoreType.DMA for _ in range(NB)]
            + [pltpu.VMEM_SHARED((P, D), jnp.float32)]
        ),
        compiler_params=pltpu.CompilerParams(needs_layout_passes=False),
    )


def _bn_body(p_ref, dinv_ref, b_ref, gamma_ref, beta_ref, out_ref):
    s = p_ref[0] + p_ref[1]
    s3 = s.reshape(P // D, D, D) * dinv_ref[...][:, :, None]
    pre = s3.reshape(P, D) + b_ref[...]
    pre = pre[:N]
    inv_n = 1.0 / float(N)
    mean = jnp.sum(pre, axis=0, keepdims=True) * inv_n
    var = jnp.sum(pre * pre, axis=0, keepdims=True) * inv_n - mean * mean
    scale = lax.rsqrt(var + EPS) * gamma_ref[...]
    out = (pre - mean) * scale + beta_ref[...]
    out_ref[...] = jnp.maximum(out, 0.0)


def _bn(parts, dinv, b, gamma, beta):
    return pl.pallas_call(
        _bn_body,
        out_shape=jax.ShapeDtypeStruct((N, D), jnp.float32),
    )(parts, dinv, b, gamma, beta)


def kernel(x, edge_index, batch, W, b, gamma, beta):
    src = edge_index[0].astype(jnp.int32)
    dst = edge_index[1].astype(jnp.int32)

    hist = _deg_kernel()(dst).reshape(NW, P // D, D)
    hp, dinv = _prescale(x, W, hist)
    zeros_blk = jnp.zeros((RPT, D), jnp.float32)
    src4 = src.reshape(NW, NBLK, IBLK, CH)
    dst4 = dst.reshape(NW, NBLK, IBLK, CH)
    parts = _scatter_kernel()(hp, src4, dst4, zeros_blk)
    return _bn(parts, dinv, b.reshape(1, D), gamma.reshape(1, D),
               beta.reshape(1, D))


# speedup vs baseline: 1.0746x; 1.0172x over previous
import functools
import jax
import jax.numpy as jnp
from jax import lax
from jax.experimental import pallas as pl
from jax.experimental.pallas import tpu as pltpu
from jax.experimental.pallas import tpu_sc as plsc

N = 10000
P = 10240
E = 320000
D = 128
EPS = 1e-5

NC, NS, L = 2, 16, 16
NW = NC * NS
EW = E // NW
CH = 40
NCH = EW // CH
IBLK = 50
NBLK = NCH // IBLK
RPT = P // NS

@functools.cache
def _mesh():
    return plsc.VectorSubcoreMesh(
        core_axis_name="c", subcore_axis_name="s", num_cores=NC, num_subcores=NS
    )


def _deg_kernel_body(dst_hbm, out_hbm, dst_v, hist_v, sem):
    cid = lax.axis_index("c")
    sid = lax.axis_index("s")
    wid = sid * NC + cid
    pltpu.async_copy(dst_hbm.at[pl.ds(wid * EW, EW)], dst_v, sem)

    def zero_body(i, _):
        hist_v[pl.ds(i * L, L)] = jnp.zeros((L,), jnp.float32)
        return 0

    lax.fori_loop(0, P // L, zero_body, 0)
    pltpu.make_async_copy(dst_hbm.at[pl.ds(wid * EW, EW)], dst_v, sem).wait()

    ones = jnp.ones((L,), jnp.float32)

    def body(i, _):
        for u in range(5):
            idx = dst_v[pl.ds((i * 5 + u) * L, L)]
            plsc.addupdate_scatter(hist_v, [idx], ones)
        return 0

    lax.fori_loop(0, EW // L // 5, body, 0)
    pltpu.sync_copy(hist_v, out_hbm.at[wid])


@functools.cache
def _deg_kernel():
    return pl.kernel(
        _deg_kernel_body,
        out_type=jax.ShapeDtypeStruct((NW, P), jnp.float32),
        mesh=_mesh(),
        scratch_types=[
            pltpu.VMEM((EW,), jnp.int32),
            pltpu.VMEM((P,), jnp.float32),
            pltpu.SemaphoreType.DMA,
        ],
        compiler_params=pltpu.CompilerParams(needs_layout_passes=False),
    )


def _prescale_body(x_ref, w_ref, hist_ref, hp_ref, dinv_ref):
    deg = jnp.sum(hist_ref[...], axis=0) + 1.0
    dinv = lax.rsqrt(deg)
    xp = jnp.concatenate(
        [x_ref[...], jnp.zeros((P - N, D), jnp.float32)], axis=0)
    h = jnp.dot(xp, w_ref[...], preferred_element_type=jnp.float32)
    h3 = h.reshape(P // D, D, D) * dinv[:, :, None]
    hp_ref[...] = h3.reshape(P, D)
    dinv_ref[...] = dinv


def _prescale(x, w, hist):
    return pl.pallas_call(
        _prescale_body,
        out_shape=(
            jax.ShapeDtypeStruct((P, D), jnp.float32),
            jax.ShapeDtypeStruct((P // D, D), jnp.float32),
        ),
    )(x, w, hist)


NB = 5


def _scatter_kernel_body(hp_hbm, src_hbm, dst_hbm, out_hbm,
                         idx_src, idx_dst, *rest):
    rows = rest[:NB]
    gsems = rest[NB:2 * NB]
    acc_sh = rest[2 * NB]
    cid = lax.axis_index("c")
    sid = lax.axis_index("s")
    wid = sid * NC + cid
    row0 = sid * RPT

    pltpu.sync_copy(src_hbm.at[wid, 0], idx_src)
    pltpu.sync_copy(dst_hbm.at[wid, 0], idx_dst)
    for j in range(1, NB):
        pltpu.async_copy(hp_hbm.at[idx_src.at[j]], rows[j], gsems[j])

    def zbody(i, _):
        rows[0][i, pl.ds(0, L)] = jnp.zeros((L,), jnp.float32)
        for u in range(1, D // L):
            rows[0][i, pl.ds(u * L, L)] = jnp.zeros((L,), jnp.float32)
        return 0

    lax.fori_loop(0, CH, zbody, 0)
    for r in range(RPT // CH):
        pltpu.sync_copy(rows[0], acc_sh.at[pl.ds(row0 + r * CH, CH)])
    pltpu.async_copy(hp_hbm.at[idx_src.at[0]], rows[0], gsems[0])
    plsc.subcore_barrier()

    def block(blk, _):
        @pl.when(blk != 0)
        def _():
            pltpu.sync_copy(src_hbm.at[wid, blk], idx_src)
            pltpu.sync_copy(dst_hbm.at[wid, blk], idx_dst)
            for j in range(NB):
                pltpu.async_copy(hp_hbm.at[idx_src.at[j]], rows[j], gsems[j])

        def group(g, _):
            for j in range(NB):
                m = g * NB + j
                pltpu.make_async_copy(hp_hbm.at[idx_src.at[m]], rows[j],
                                      gsems[j]).wait()
                pltpu.sync_copy(rows[j], acc_sh.at[idx_dst.at[m]], add=True)
                n = m + NB

                @pl.when(n < IBLK)
                def _():
                    pltpu.async_copy(hp_hbm.at[idx_src.at[n]], rows[j],
                                     gsems[j])
            return 0

        lax.fori_loop(0, IBLK // NB, group, 0)
        return 0

    lax.fori_loop(0, NBLK, block, 0)
    plsc.subcore_barrier()

    pltpu.sync_copy(acc_sh.at[pl.ds(sid * RPT, RPT)],
                    out_hbm.at[cid, pl.ds(sid * RPT, RPT)])


@functools.cache
def _scatter_kernel():
    return pl.kernel(
        _scatter_kernel_body,
        out_type=jax.ShapeDtypeStruct((NC, P, D), jnp.float32),
        mesh=_mesh(),
        scratch_types=(
            [pltpu.VMEM((IBLK, CH), jnp.int32),
             pltpu.VMEM((IBLK, CH), jnp.int32)]
            + [pltpu.VMEM((CH, D), jnp.float32) for _ in range(NB)]
            + [pltpu.SemaphoreType.DMA for _ in range(NB)]
            + [pltpu.VMEM_SHARED((P, D), jnp.float32)]
        ),
        compiler_params=pltpu.CompilerParams(needs_layout_passes=False),
    )


def _bn_body(hp_ref, p_ref, dinv_ref, b_ref, gamma_ref, beta_ref, out_ref):
    s = hp_ref[...] + p_ref[0] + p_ref[1]
    s3 = s.reshape(P // D, D, D) * dinv_ref[...][:, :, None]
    pre = s3.reshape(P, D) + b_ref[...]
    pre = pre[:N]
    inv_n = 1.0 / float(N)
    mean = jnp.sum(pre, axis=0, keepdims=True) * inv_n
    var = jnp.sum(pre * pre, axis=0, keepdims=True) * inv_n - mean * mean
    scale = lax.rsqrt(var + EPS) * gamma_ref[...]
    out = (pre - mean) * scale + beta_ref[...]
    out_ref[...] = jnp.maximum(out, 0.0)


def _bn(hp, parts, dinv, b, gamma, beta):
    return pl.pallas_call(
        _bn_body,
        out_shape=jax.ShapeDtypeStruct((N, D), jnp.float32),
    )(hp, parts, dinv, b, gamma, beta)


def kernel(x, edge_index, batch, W, b, gamma, beta):
    src = edge_index[0].astype(jnp.int32)
    dst = edge_index[1].astype(jnp.int32)

    hist = _deg_kernel()(dst).reshape(NW, P // D, D)
    hp, dinv = _prescale(x, W, hist)
    src4 = src.reshape(NW, NBLK, IBLK, CH)
    dst4 = dst.reshape(NW, NBLK, IBLK, CH)
    parts = _scatter_kernel()(hp, src4, dst4)
    return _bn(hp, parts, dinv, b.reshape(1, D), gamma.reshape(1, D),
               beta.reshape(1, D))
